# baseline (speedup 1.0000x reference)
import functools

import jax
import jax.numpy as jnp
from jax import lax
from jax.experimental import pallas as pl
from jax.experimental.pallas import tpu as pltpu
from jax.experimental.pallas import tpu_sc as plsc

DIM = 1024
INTER = 512
E = 64
TOP_K = 2
BLK = 256
NTOK = 8192
LPAD = NTOK * TOP_K + E * BLK
NB = LPAD // BLK
HDIM = DIM // 2
GATE_BLK = 512
F32_MIN = float(jnp.finfo(jnp.float32).min)


def _gate_body(x_ref, wg_ref, idx_ref, wgt_ref, rank_ref, cnt_ref,
               xb_ref, acc_ref):
    @pl.when(pl.program_id(0) == 0)
    def _():
        acc_ref[...] = jnp.zeros_like(acc_ref)

    x = x_ref[...]
    logits = lax.dot_general(x, wg_ref[...], (((1,), (1,)), ((), ())),
                             preferred_element_type=jnp.float32)
    m = jnp.max(logits, axis=-1, keepdims=True)
    ex = jnp.exp(logits - m)
    scores = ex / jnp.sum(ex, axis=-1, keepdims=True)
    cols = lax.broadcasted_iota(jnp.int32, scores.shape, 1)
    m1 = jnp.max(scores, axis=-1, keepdims=True)
    a1 = jnp.min(jnp.where(scores == m1, cols, E), axis=-1, keepdims=True)
    s2 = jnp.where(cols == a1, F32_MIN, scores)
    m2 = jnp.max(s2, axis=-1, keepdims=True)
    a2 = jnp.min(jnp.where(s2 == m2, cols, E), axis=-1, keepdims=True)

    oh0 = (cols == a1).astype(jnp.float32)
    oh1 = (cols == a2).astype(jnp.float32)
    both = oh0 + oh1
    r_i = lax.broadcasted_iota(jnp.int32, (GATE_BLK, GATE_BLK), 0)
    c_i = lax.broadcasted_iota(jnp.int32, (GATE_BLK, GATE_BLK), 1)
    lower = (r_i > c_i).astype(jnp.float32)
    prior = lax.dot_general(lower, both, (((1,), (0,)), ((), ())),
                            preferred_element_type=jnp.float32)
    prior = prior + acc_ref[...]
    rank0 = jnp.sum(oh0 * prior, axis=1, keepdims=True)
    rank1 = jnp.sum(oh1 * prior, axis=1, keepdims=True)
    acc_new = acc_ref[...] + jnp.sum(both, axis=0, keepdims=True)
    acc_ref[...] = acc_new

    xb_ref[...] = x.astype(jnp.bfloat16)
    idx_ref[...] = jnp.concatenate([a1, a2], axis=1)
    wgt_ref[...] = jnp.concatenate([m1, m2], axis=1)
    rank_ref[...] = jnp.concatenate([rank0, rank1], axis=1).astype(jnp.int32)
    cnt_ref[...] = acc_new.astype(jnp.int32)


def _gate(x2, Wg):
    n = x2.shape[0]
    return pl.pallas_call(
        _gate_body,
        grid=(n // GATE_BLK,),
        in_specs=[
            pl.BlockSpec((GATE_BLK, DIM), lambda i: (i, 0)),
            pl.BlockSpec((E, DIM), lambda i: (0, 0)),
        ],
        out_specs=[
            pl.BlockSpec((GATE_BLK, TOP_K), lambda i: (i, 0)),
            pl.BlockSpec((GATE_BLK, TOP_K), lambda i: (i, 0)),
            pl.BlockSpec((GATE_BLK, TOP_K), lambda i: (i, 0)),
            pl.BlockSpec((1, E), lambda i: (0, 0)),
            pl.BlockSpec((GATE_BLK, DIM), lambda i: (i, 0)),
        ],
        out_shape=[
            jax.ShapeDtypeStruct((n, TOP_K), jnp.int32),
            jax.ShapeDtypeStruct((n, TOP_K), jnp.float32),
            jax.ShapeDtypeStruct((n, TOP_K), jnp.int32),
            jax.ShapeDtypeStruct((1, E), jnp.int32),
            jax.ShapeDtypeStruct((n, DIM), jnp.bfloat16),
        ],
        scratch_shapes=[pltpu.VMEM((1, E), jnp.float32)],
    )(x2, Wg)


def _slot_body(idx_ref, rank_ref, cnt_ref, slot_ref, be_ref, val_ref):
    cnt = cnt_ref[...].astype(jnp.float32)
    pc = jnp.floor((cnt + (BLK - 1)) * (1.0 / BLK)) * BLK
    r_i = lax.broadcasted_iota(jnp.int32, (E, E), 0)
    c_i = lax.broadcasted_iota(jnp.int32, (E, E), 1)
    upper = (r_i < c_i).astype(jnp.float32)
    pstart = lax.dot_general(pc, upper, (((1,), (0,)), ((), ())),
                             preferred_element_type=jnp.float32)
    cols = lax.broadcasted_iota(jnp.int32, (NTOK, E), 1)
    oh0 = (cols == idx_ref[:, 0:1]).astype(jnp.float32)
    oh1 = (cols == idx_ref[:, 1:2]).astype(jnp.float32)
    s0 = lax.dot_general(oh0, pstart, (((1,), (1,)), ((), ())),
                         preferred_element_type=jnp.float32)
    s1 = lax.dot_general(oh1, pstart, (((1,), (1,)), ((), ())),
                         preferred_element_type=jnp.float32)
    base = jnp.concatenate([s0, s1], axis=1)
    slot_ref[...] = base.astype(jnp.int32) + rank_ref[...]

    eye = (r_i == c_i).astype(jnp.float32)
    strict = (r_i > c_i).astype(jnp.float32)
    incl = (r_i >= c_i).astype(jnp.float32)
    pad_cum_t = lax.dot_general(incl, pc, (((1,), (1,)), ((), ())),
                                preferred_element_type=jnp.float32)
    pstart_t = lax.dot_general(strict, pc, (((1,), (1,)), ((), ())),
                               preferred_element_type=jnp.float32)
    cnt_t = lax.dot_general(eye, cnt, (((1,), (1,)), ((), ())),
                            preferred_element_type=jnp.float32)
    realend_t = pstart_t + cnt_t
    b_off = (lax.broadcasted_iota(jnp.int32, (E, NB), 1)
             * BLK).astype(jnp.float32)
    eid_f = jnp.sum((pad_cum_t <= b_off).astype(jnp.float32),
                    axis=0, keepdims=True)
    val_f = jnp.sum(((pstart_t <= b_off) & (b_off < realend_t))
                    .astype(jnp.float32), axis=0, keepdims=True)
    be_ref[...] = jnp.minimum(eid_f, float(E - 1)).astype(jnp.int32)
    val_ref[...] = val_f.astype(jnp.int32)


def _slots(idx, rank, cnt):
    return pl.pallas_call(
        _slot_body,
        grid=(1,),
        in_specs=[
            pl.BlockSpec((NTOK, TOP_K), lambda i: (0, 0)),
            pl.BlockSpec((NTOK, TOP_K), lambda i: (0, 0)),
            pl.BlockSpec((1, E), lambda i: (0, 0)),
        ],
        out_specs=[
            pl.BlockSpec((NTOK, TOP_K), lambda i: (0, 0)),
            pl.BlockSpec((1, NB), lambda i: (0, 0)),
            pl.BlockSpec((1, NB), lambda i: (0, 0)),
        ],
        out_shape=[
            jax.ShapeDtypeStruct((NTOK, TOP_K), jnp.int32),
            jax.ShapeDtypeStruct((1, NB), jnp.int32),
            jax.ShapeDtypeStruct((1, NB), jnp.int32),
        ],
    )(idx, rank, cnt)


_SC_CH = 64


def _iota16(base):
    return lax.iota(jnp.int32, 16) + base


def _sc_dispatch(x2, slot):
    na = NTOK * TOP_K
    info = plsc.get_sparse_core_info()
    nw = info.num_cores * info.num_subcores
    per_w = na // nw
    steps = per_w // _SC_CH
    mesh = plsc.VectorSubcoreMesh(core_axis_name="c", subcore_axis_name="s")

    @functools.partial(
        pl.kernel,
        mesh=mesh,
        out_type=jax.ShapeDtypeStruct((LPAD, HDIM), jnp.int32),
        scratch_types=[
            pltpu.VMEM((_SC_CH,), jnp.int32),
            pltpu.VMEM((_SC_CH,), jnp.int32),
            pltpu.VMEM((_SC_CH, HDIM), jnp.int32),
            pltpu.SemaphoreType.DMA,
            pltpu.SemaphoreType.DMA,
        ],
    )
    def dispatch_k(x_hbm, slot_hbm, xs_hbm, sv, tok_v, rows_v, s0, s1):
        wid = lax.axis_index("s") * info.num_cores + lax.axis_index("c")
        base = wid * per_w

        def body(i, carry):
            aoff = base + i * _SC_CH
            pltpu.sync_copy(slot_hbm.at[pl.ds(aoff, _SC_CH)], sv)
            for b in range(_SC_CH // 16):
                j = _iota16(aoff + 16 * b)
                tok_v[pl.ds(16 * b, 16)] = lax.shift_right_logical(j, 1)
            pltpu.async_copy(x_hbm.at[tok_v], rows_v, s0).wait()
            pltpu.async_copy(rows_v, xs_hbm.at[sv], s1).wait()
            return carry

        lax.fori_loop(0, steps, body, 0)

    return dispatch_k(x2, slot)


def _sc_combine(ys, slot):
    na = NTOK * TOP_K
    info = plsc.get_sparse_core_info()
    nw = info.num_cores * info.num_subcores
    per_w = na // nw
    steps = per_w // _SC_CH
    mesh = plsc.VectorSubcoreMesh(core_axis_name="c", subcore_axis_name="s")

    @functools.partial(
        pl.kernel,
        mesh=mesh,
        out_type=jax.ShapeDtypeStruct((TOP_K * NTOK, HDIM), jnp.int32),
        scratch_types=[
            pltpu.VMEM((_SC_CH,), jnp.int32),
            pltpu.VMEM((_SC_CH,), jnp.int32),
            pltpu.VMEM((_SC_CH, HDIM), jnp.int32),
            pltpu.SemaphoreType.DMA,
            pltpu.SemaphoreType.DMA,
        ],
    )
    def combine_k(ys_hbm, slot_hbm, g_hbm, sv, dst_v, rows_v, s0, s1):
        wid = lax.axis_index("s") * info.num_cores + lax.axis_index("c")
        base = wid * per_w

        def body(i, carry):
            aoff = base + i * _SC_CH
            pltpu.sync_copy(slot_hbm.at[pl.ds(aoff, _SC_CH)], sv)
            for b in range(_SC_CH // 16):
                j = _iota16(aoff + 16 * b)
                dst_v[pl.ds(16 * b, 16)] = (
                    lax.shift_right_logical(j, 1)
                    + (j & 1) * NTOK)
            pltpu.async_copy(ys_hbm.at[sv], rows_v, s0).wait()
            pltpu.async_copy(rows_v, g_hbm.at[dst_v], s1).wait()
            return carry

        lax.fori_loop(0, steps, body, 0)

    return combine_k(ys, slot)


def _group_body(eid_ref, val_ref, xs_ref, w1_ref, w3_ref, w2_ref, ys_ref):
    @pl.when(val_ref[pl.program_id(0)] == 1)
    def _():
        x = xs_ref[...]
        w1b = w1_ref[0].astype(jnp.bfloat16)
        w3b = w3_ref[0].astype(jnp.bfloat16)
        h1 = lax.dot_general(x, w1b, (((1,), (1,)), ((), ())),
                             preferred_element_type=jnp.float32)
        h3 = lax.dot_general(x, w3b, (((1,), (1,)), ((), ())),
                             preferred_element_type=jnp.float32)
        h = ((h1 * jax.nn.sigmoid(h1)) * h3).astype(jnp.bfloat16)
        w2b = w2_ref[0].astype(jnp.bfloat16)
        y = lax.dot_general(h, w2b, (((1,), (1,)), ((), ())),
                            preferred_element_type=jnp.float32)
        ys_ref[...] = y.astype(jnp.bfloat16)


def _grouped_mlp(xs, w1, w2, w3, blk_eid, valid):
    grid_spec = pltpu.PrefetchScalarGridSpec(
        num_scalar_prefetch=2,
        grid=(NB,),
        in_specs=[
            pl.BlockSpec((BLK, DIM), lambda i, e, v: (i, 0)),
            pl.BlockSpec((1, INTER, DIM), lambda i, e, v: (e[i], 0, 0)),
            pl.BlockSpec((1, INTER, DIM), lambda i, e, v: (e[i], 0, 0)),
            pl.BlockSpec((1, DIM, INTER), lambda i, e, v: (e[i], 0, 0)),
        ],
        out_specs=pl.BlockSpec((BLK, DIM), lambda i, e, v: (i, 0)),
    )
    return pl.pallas_call(
        _group_body,
        grid_spec=grid_spec,
        out_shape=jax.ShapeDtypeStruct((LPAD, DIM), jnp.bfloat16),
    )(blk_eid, valid, xs, w1, w3, w2)


def _final_body(x_ref, sw1_ref, sw3_ref, sw2_ref, g_ref, wgt_ref, out_ref):
    x = x_ref[...]
    h1 = lax.dot_general(x, sw1_ref[...], (((1,), (1,)), ((), ())),
                         preferred_element_type=jnp.float32)
    h3 = lax.dot_general(x, sw3_ref[...], (((1,), (1,)), ((), ())),
                         preferred_element_type=jnp.float32)
    h = (h1 * jax.nn.sigmoid(h1)) * h3
    z = lax.dot_general(h, sw2_ref[...], (((1,), (1,)), ((), ())),
                        preferred_element_type=jnp.float32)
    w0 = wgt_ref[:, 0:1]
    w1c = wgt_ref[:, 1:2]
    g0 = g_ref[0].astype(jnp.float32)
    g1 = g_ref[1].astype(jnp.float32)
    out_ref[...] = z + w0 * g0 + w1c * g1


def _final(x2, sw1, sw2, sw3, garr, wgt):
    n = x2.shape[0]
    return pl.pallas_call(
        _final_body,
        grid=(n // GATE_BLK,),
        in_specs=[
            pl.BlockSpec((GATE_BLK, DIM), lambda i: (i, 0)),
            pl.BlockSpec((INTER, DIM), lambda i: (0, 0)),
            pl.BlockSpec((INTER, DIM), lambda i: (0, 0)),
            pl.BlockSpec((DIM, INTER), lambda i: (0, 0)),
            pl.BlockSpec((TOP_K, GATE_BLK, DIM), lambda i: (0, i, 0)),
            pl.BlockSpec((GATE_BLK, TOP_K), lambda i: (i, 0)),
        ],
        out_specs=pl.BlockSpec((GATE_BLK, DIM), lambda i: (i, 0)),
        out_shape=jax.ShapeDtypeStruct((n, DIM), jnp.float32),
    )(x2, sw1, sw3, sw2, garr, wgt)


def kernel(x, Wg, w1, w2, w3, sw1, sw2, sw3):
    shape = x.shape
    x2 = x.reshape(-1, DIM)

    idx, wgt, rank, cnt, xb = _gate(x2, Wg)
    slot, be2, val2 = _slots(idx, rank, cnt)
    blk_eid = be2.reshape(NB)
    valid = val2.reshape(NB)

    slot_flat = slot.reshape(-1)
    xb32 = lax.bitcast_convert_type(
        xb.reshape(NTOK, HDIM, 2), jnp.int32)
    xs32 = _sc_dispatch(xb32, slot_flat)
    xs = lax.bitcast_convert_type(xs32, jnp.bfloat16).reshape(LPAD, DIM)
    ys = _grouped_mlp(xs, w1, w2, w3, blk_eid, valid)
    ys32 = lax.bitcast_convert_type(
        ys.reshape(LPAD, HDIM, 2), jnp.int32)
    garr32 = _sc_combine(ys32, slot_flat)
    garr = lax.bitcast_convert_type(garr32, jnp.bfloat16).reshape(
        TOP_K, NTOK, DIM)
    out = _final(x2, sw1, sw2, sw3, garr, wgt)
    return out.reshape(shape)


# speedup vs baseline: 3.7587x; 3.7587x over previous
import functools

import jax
import jax.numpy as jnp
from jax import lax
from jax.experimental import pallas as pl
from jax.experimental.pallas import tpu as pltpu
from jax.experimental.pallas import tpu_sc as plsc

DIM = 1024
INTER = 512
E = 64
TOP_K = 2
BLK = 256
NTOK = 8192
LPAD = NTOK * TOP_K + E * BLK
NB = LPAD // BLK
GATE_BLK = 512
F32_MIN = float(jnp.finfo(jnp.float32).min)


def _gate_body(x_ref, wg_ref, idx_ref, wgt_ref, rank_ref, cnt_ref, acc_ref):
    @pl.when(pl.program_id(0) == 0)
    def _():
        acc_ref[...] = jnp.zeros_like(acc_ref)

    x = x_ref[...]
    logits = lax.dot_general(x, wg_ref[...], (((1,), (1,)), ((), ())),
                             preferred_element_type=jnp.float32)
    m = jnp.max(logits, axis=-1, keepdims=True)
    ex = jnp.exp(logits - m)
    scores = ex / jnp.sum(ex, axis=-1, keepdims=True)
    cols = lax.broadcasted_iota(jnp.int32, scores.shape, 1)
    m1 = jnp.max(scores, axis=-1, keepdims=True)
    a1 = jnp.min(jnp.where(scores == m1, cols, E), axis=-1, keepdims=True)
    s2 = jnp.where(cols == a1, F32_MIN, scores)
    m2 = jnp.max(s2, axis=-1, keepdims=True)
    a2 = jnp.min(jnp.where(s2 == m2, cols, E), axis=-1, keepdims=True)

    oh0 = (cols == a1).astype(jnp.float32)
    oh1 = (cols == a2).astype(jnp.float32)
    both = oh0 + oh1
    r_i = lax.broadcasted_iota(jnp.int32, (GATE_BLK, GATE_BLK), 0)
    c_i = lax.broadcasted_iota(jnp.int32, (GATE_BLK, GATE_BLK), 1)
    lower = (r_i > c_i).astype(jnp.float32)
    prior = lax.dot_general(lower, both, (((1,), (0,)), ((), ())),
                            preferred_element_type=jnp.float32)
    prior = prior + acc_ref[...]
    rank0 = jnp.sum(oh0 * prior, axis=1, keepdims=True)
    rank1 = jnp.sum(oh1 * prior, axis=1, keepdims=True)
    acc_new = acc_ref[...] + jnp.sum(both, axis=0, keepdims=True)
    acc_ref[...] = acc_new

    idx_ref[...] = jnp.concatenate([a1, a2], axis=1)
    wgt_ref[...] = jnp.concatenate([m1, m2], axis=1)
    rank_ref[...] = jnp.concatenate([rank0, rank1], axis=1).astype(jnp.int32)
    cnt_ref[...] = acc_new.astype(jnp.int32)


def _gate(x2, Wg):
    n = x2.shape[0]
    return pl.pallas_call(
        _gate_body,
        grid=(n // GATE_BLK,),
        in_specs=[
            pl.BlockSpec((GATE_BLK, DIM), lambda i: (i, 0)),
            pl.BlockSpec((E, DIM), lambda i: (0, 0)),
        ],
        out_specs=[
            pl.BlockSpec((GATE_BLK, TOP_K), lambda i: (i, 0)),
            pl.BlockSpec((GATE_BLK, TOP_K), lambda i: (i, 0)),
            pl.BlockSpec((GATE_BLK, TOP_K), lambda i: (i, 0)),
            pl.BlockSpec((1, E), lambda i: (0, 0)),
        ],
        out_shape=[
            jax.ShapeDtypeStruct((n, TOP_K), jnp.int32),
            jax.ShapeDtypeStruct((n, TOP_K), jnp.float32),
            jax.ShapeDtypeStruct((n, TOP_K), jnp.int32),
            jax.ShapeDtypeStruct((1, E), jnp.int32),
        ],
        scratch_shapes=[pltpu.VMEM((1, E), jnp.float32)],
    )(x2, Wg)


def _slot_body(idx_ref, rank_ref, cnt_ref, slot_ref, be_ref, val_ref):
    cnt = cnt_ref[...].astype(jnp.float32)
    pc = jnp.floor((cnt + (BLK - 1)) * (1.0 / BLK)) * BLK
    r_i = lax.broadcasted_iota(jnp.int32, (E, E), 0)
    c_i = lax.broadcasted_iota(jnp.int32, (E, E), 1)
    upper = (r_i < c_i).astype(jnp.float32)
    pstart = lax.dot_general(pc, upper, (((1,), (0,)), ((), ())),
                             preferred_element_type=jnp.float32)
    cols = lax.broadcasted_iota(jnp.int32, (NTOK, E), 1)
    oh0 = (cols == idx_ref[:, 0:1]).astype(jnp.float32)
    oh1 = (cols == idx_ref[:, 1:2]).astype(jnp.float32)
    s0 = lax.dot_general(oh0, pstart, (((1,), (1,)), ((), ())),
                         preferred_element_type=jnp.float32)
    s1 = lax.dot_general(oh1, pstart, (((1,), (1,)), ((), ())),
                         preferred_element_type=jnp.float32)
    base = jnp.concatenate([s0, s1], axis=1)
    slot_ref[...] = base.astype(jnp.int32) + rank_ref[...]

    eye = (r_i == c_i).astype(jnp.float32)
    strict = (r_i > c_i).astype(jnp.float32)
    incl = (r_i >= c_i).astype(jnp.float32)
    pad_cum_t = lax.dot_general(incl, pc, (((1,), (1,)), ((), ())),
                                preferred_element_type=jnp.float32)
    pstart_t = lax.dot_general(strict, pc, (((1,), (1,)), ((), ())),
                               preferred_element_type=jnp.float32)
    cnt_t = lax.dot_general(eye, cnt, (((1,), (1,)), ((), ())),
                            preferred_element_type=jnp.float32)
    realend_t = pstart_t + cnt_t
    b_off = (lax.broadcasted_iota(jnp.int32, (E, NB), 1)
             * BLK).astype(jnp.float32)
    eid_f = jnp.sum((pad_cum_t <= b_off).astype(jnp.float32),
                    axis=0, keepdims=True)
    val_f = jnp.sum(((pstart_t <= b_off) & (b_off < realend_t))
                    .astype(jnp.float32), axis=0, keepdims=True)
    be_ref[...] = jnp.minimum(eid_f, float(E - 1)).astype(jnp.int32)
    val_ref[...] = val_f.astype(jnp.int32)


def _slots(idx, rank, cnt):
    return pl.pallas_call(
        _slot_body,
        grid=(1,),
        in_specs=[
            pl.BlockSpec((NTOK, TOP_K), lambda i: (0, 0)),
            pl.BlockSpec((NTOK, TOP_K), lambda i: (0, 0)),
            pl.BlockSpec((1, E), lambda i: (0, 0)),
        ],
        out_specs=[
            pl.BlockSpec((NTOK, TOP_K), lambda i: (0, 0)),
            pl.BlockSpec((1, NB), lambda i: (0, 0)),
            pl.BlockSpec((1, NB), lambda i: (0, 0)),
        ],
        out_shape=[
            jax.ShapeDtypeStruct((NTOK, TOP_K), jnp.int32),
            jax.ShapeDtypeStruct((1, NB), jnp.int32),
            jax.ShapeDtypeStruct((1, NB), jnp.int32),
        ],
    )(idx, rank, cnt)


_SC_CH = 64


def _iota16(base):
    return lax.iota(jnp.int32, 16) + base


def _sc_dispatch(x2, slot):
    na = NTOK * TOP_K
    info = plsc.get_sparse_core_info()
    nw = info.num_cores * info.num_subcores
    per_w = na // nw
    steps = per_w // _SC_CH
    mesh = plsc.VectorSubcoreMesh(core_axis_name="c", subcore_axis_name="s")

    @functools.partial(
        pl.kernel,
        mesh=mesh,
        out_type=jax.ShapeDtypeStruct((LPAD, DIM), jnp.float32),
        scratch_types=[
            pltpu.VMEM((_SC_CH,), jnp.int32),
            pltpu.VMEM((_SC_CH,), jnp.int32),
            pltpu.VMEM((_SC_CH, DIM), jnp.float32),
            pltpu.SemaphoreType.DMA,
            pltpu.SemaphoreType.DMA,
        ],
    )
    def dispatch_k(x_hbm, slot_hbm, xs_hbm, sv, tok_v, rows_v, s0, s1):
        wid = lax.axis_index("s") * info.num_cores + lax.axis_index("c")
        base = wid * per_w

        def body(i, carry):
            aoff = base + i * _SC_CH
            pltpu.sync_copy(slot_hbm.at[pl.ds(aoff, _SC_CH)], sv)
            for b in range(_SC_CH // 16):
                j = _iota16(aoff + 16 * b)
                tok_v[pl.ds(16 * b, 16)] = lax.shift_right_logical(j, 1)
            pltpu.async_copy(x_hbm.at[tok_v], rows_v, s0).wait()
            pltpu.async_copy(rows_v, xs_hbm.at[sv], s1).wait()
            return carry

        lax.fori_loop(0, steps, body, 0)

    return dispatch_k(x2, slot)


def _sc_combine(ys, slot):
    na = NTOK * TOP_K
    info = plsc.get_sparse_core_info()
    nw = info.num_cores * info.num_subcores
    per_w = na // nw
    steps = per_w // _SC_CH
    mesh = plsc.VectorSubcoreMesh(core_axis_name="c", subcore_axis_name="s")

    @functools.partial(
        pl.kernel,
        mesh=mesh,
        out_type=jax.ShapeDtypeStruct((TOP_K * NTOK, DIM), jnp.float32),
        scratch_types=[
            pltpu.VMEM((_SC_CH,), jnp.int32),
            pltpu.VMEM((_SC_CH,), jnp.int32),
            pltpu.VMEM((_SC_CH, DIM), jnp.float32),
            pltpu.SemaphoreType.DMA,
            pltpu.SemaphoreType.DMA,
        ],
    )
    def combine_k(ys_hbm, slot_hbm, g_hbm, sv, dst_v, rows_v, s0, s1):
        wid = lax.axis_index("s") * info.num_cores + lax.axis_index("c")
        base = wid * per_w

        def body(i, carry):
            aoff = base + i * _SC_CH
            pltpu.sync_copy(slot_hbm.at[pl.ds(aoff, _SC_CH)], sv)
            for b in range(_SC_CH // 16):
                j = _iota16(aoff + 16 * b)
                dst_v[pl.ds(16 * b, 16)] = (
                    lax.shift_right_logical(j, 1)
                    + (j & 1) * NTOK)
            pltpu.async_copy(ys_hbm.at[sv], rows_v, s0).wait()
            pltpu.async_copy(rows_v, g_hbm.at[dst_v], s1).wait()
            return carry

        lax.fori_loop(0, steps, body, 0)

    return combine_k(ys, slot)


def _group_body(eid_ref, val_ref, xs_ref, w1_ref, w3_ref, w2_ref, ys_ref):
    @pl.when(val_ref[pl.program_id(0)] == 1)
    def _():
        x = xs_ref[...].astype(jnp.bfloat16)
        w1b = w1_ref[0].astype(jnp.bfloat16)
        w3b = w3_ref[0].astype(jnp.bfloat16)
        h1 = lax.dot_general(x, w1b, (((1,), (1,)), ((), ())),
                             preferred_element_type=jnp.float32)
        h3 = lax.dot_general(x, w3b, (((1,), (1,)), ((), ())),
                             preferred_element_type=jnp.float32)
        h = ((h1 * jax.nn.sigmoid(h1)) * h3).astype(jnp.bfloat16)
        w2b = w2_ref[0].astype(jnp.bfloat16)
        ys_ref[...] = lax.dot_general(h, w2b, (((1,), (1,)), ((), ())),
                                      preferred_element_type=jnp.float32)


def _grouped_mlp(xs, w1, w2, w3, blk_eid, valid):
    grid_spec = pltpu.PrefetchScalarGridSpec(
        num_scalar_prefetch=2,
        grid=(NB,),
        in_specs=[
            pl.BlockSpec((BLK, DIM), lambda i, e, v: (i, 0)),
            pl.BlockSpec((1, INTER, DIM), lambda i, e, v: (e[i], 0, 0)),
            pl.BlockSpec((1, INTER, DIM), lambda i, e, v: (e[i], 0, 0)),
            pl.BlockSpec((1, DIM, INTER), lambda i, e, v: (e[i], 0, 0)),
        ],
        out_specs=pl.BlockSpec((BLK, DIM), lambda i, e, v: (i, 0)),
    )
    return pl.pallas_call(
        _group_body,
        grid_spec=grid_spec,
        out_shape=jax.ShapeDtypeStruct((LPAD, DIM), jnp.float32),
    )(blk_eid, valid, xs, w1, w3, w2)


def _shared_body(x_ref, sw1_ref, sw3_ref, sw2_ref, z_ref):
    x = x_ref[...]
    h1 = lax.dot_general(x, sw1_ref[...], (((1,), (1,)), ((), ())),
                         preferred_element_type=jnp.float32)
    h3 = lax.dot_general(x, sw3_ref[...], (((1,), (1,)), ((), ())),
                         preferred_element_type=jnp.float32)
    h = (h1 * jax.nn.sigmoid(h1)) * h3
    z_ref[...] = lax.dot_general(h, sw2_ref[...], (((1,), (1,)), ((), ())),
                                 preferred_element_type=jnp.float32)


def _shared(x2, sw1, sw2, sw3):
    n = x2.shape[0]
    return pl.pallas_call(
        _shared_body,
        grid=(n // GATE_BLK,),
        in_specs=[
            pl.BlockSpec((GATE_BLK, DIM), lambda i: (i, 0)),
            pl.BlockSpec((INTER, DIM), lambda i: (0, 0)),
            pl.BlockSpec((INTER, DIM), lambda i: (0, 0)),
            pl.BlockSpec((DIM, INTER), lambda i: (0, 0)),
        ],
        out_specs=pl.BlockSpec((GATE_BLK, DIM), lambda i: (i, 0)),
        out_shape=jax.ShapeDtypeStruct((n, DIM), jnp.float32),
    )(x2, sw1, sw3, sw2)


def _final_body(z_ref, g_ref, wgt_ref, out_ref):
    w0 = wgt_ref[:, 0:1]
    w1c = wgt_ref[:, 1:2]
    out_ref[...] = z_ref[...] + w0 * g_ref[0] + w1c * g_ref[1]


def _final(z, garr, wgt):
    n = z.shape[0]
    return pl.pallas_call(
        _final_body,
        grid=(n // GATE_BLK,),
        in_specs=[
            pl.BlockSpec((GATE_BLK, DIM), lambda i: (i, 0)),
            pl.BlockSpec((TOP_K, GATE_BLK, DIM), lambda i: (0, i, 0)),
            pl.BlockSpec((GATE_BLK, TOP_K), lambda i: (i, 0)),
        ],
        out_specs=pl.BlockSpec((GATE_BLK, DIM), lambda i: (i, 0)),
        out_shape=jax.ShapeDtypeStruct((n, DIM), jnp.float32),
    )(z, garr, wgt)


def kernel(x, Wg, w1, w2, w3, sw1, sw2, sw3):
    shape = x.shape
    x2 = x.reshape(-1, DIM)

    idx, wgt, rank, cnt = _gate(x2, Wg)
    slot, be2, val2 = _slots(idx, rank, cnt)
    blk_eid = be2.reshape(NB)
    valid = val2.reshape(NB)

    slot_flat = slot.reshape(-1)
    z = _shared(x2, sw1, sw2, sw3)
    xs = _sc_dispatch(x2, slot_flat)
    ys = _grouped_mlp(xs, w1, w2, w3, blk_eid, valid)
    garr = _sc_combine(ys, slot_flat).reshape(TOP_K, NTOK, DIM)
    out = _final(z, garr, wgt)
    return out.reshape(shape)


# speedup vs baseline: 3.8155x; 1.0151x over previous
import functools

import jax
import jax.numpy as jnp
from jax import lax
from jax.experimental import pallas as pl
from jax.experimental.pallas import tpu as pltpu
from jax.experimental.pallas import tpu_sc as plsc

DIM = 1024
INTER = 512
E = 64
TOP_K = 2
BLK = 256
NTOK = 8192
LPAD = NTOK * TOP_K + E * BLK
NB = LPAD // BLK
GATE_BLK = 512
F32_MIN = float(jnp.finfo(jnp.float32).min)


def _gate_body(x_ref, wg_ref, idx_ref, wgt_ref, rank_ref, cnt_ref, acc_ref):
    @pl.when(pl.program_id(0) == 0)
    def _():
        acc_ref[...] = jnp.zeros_like(acc_ref)

    x = x_ref[...]
    logits = lax.dot_general(x, wg_ref[...], (((1,), (1,)), ((), ())),
                             preferred_element_type=jnp.float32)
    m = jnp.max(logits, axis=-1, keepdims=True)
    ex = jnp.exp(logits - m)
    scores = ex / jnp.sum(ex, axis=-1, keepdims=True)
    cols = lax.broadcasted_iota(jnp.int32, scores.shape, 1)
    m1 = jnp.max(scores, axis=-1, keepdims=True)
    a1 = jnp.min(jnp.where(scores == m1, cols, E), axis=-1, keepdims=True)
    s2 = jnp.where(cols == a1, F32_MIN, scores)
    m2 = jnp.max(s2, axis=-1, keepdims=True)
    a2 = jnp.min(jnp.where(s2 == m2, cols, E), axis=-1, keepdims=True)

    oh0 = (cols == a1).astype(jnp.float32)
    oh1 = (cols == a2).astype(jnp.float32)
    both = oh0 + oh1
    r_i = lax.broadcasted_iota(jnp.int32, (GATE_BLK, GATE_BLK), 0)
    c_i = lax.broadcasted_iota(jnp.int32, (GATE_BLK, GATE_BLK), 1)
    lower = (r_i > c_i).astype(jnp.float32)
    prior = lax.dot_general(lower, both, (((1,), (0,)), ((), ())),
                            preferred_element_type=jnp.float32)
    prior = prior + acc_ref[...]
    rank0 = jnp.sum(oh0 * prior, axis=1, keepdims=True)
    rank1 = jnp.sum(oh1 * prior, axis=1, keepdims=True)
    acc_new = acc_ref[...] + jnp.sum(both, axis=0, keepdims=True)
    acc_ref[...] = acc_new

    idx_ref[...] = jnp.concatenate([a1, a2], axis=1)
    wgt_ref[...] = jnp.concatenate([m1, m2], axis=1)
    rank_ref[...] = jnp.concatenate([rank0, rank1], axis=1).astype(jnp.int32)
    cnt_ref[...] = acc_new.astype(jnp.int32)


def _gate(x2, Wg):
    n = x2.shape[0]
    return pl.pallas_call(
        _gate_body,
        grid=(n // GATE_BLK,),
        in_specs=[
            pl.BlockSpec((GATE_BLK, DIM), lambda i: (i, 0)),
            pl.BlockSpec((E, DIM), lambda i: (0, 0)),
        ],
        out_specs=[
            pl.BlockSpec((GATE_BLK, TOP_K), lambda i: (i, 0)),
            pl.BlockSpec((GATE_BLK, TOP_K), lambda i: (i, 0)),
            pl.BlockSpec((GATE_BLK, TOP_K), lambda i: (i, 0)),
            pl.BlockSpec((1, E), lambda i: (0, 0)),
        ],
        out_shape=[
            jax.ShapeDtypeStruct((n, TOP_K), jnp.int32),
            jax.ShapeDtypeStruct((n, TOP_K), jnp.float32),
            jax.ShapeDtypeStruct((n, TOP_K), jnp.int32),
            jax.ShapeDtypeStruct((1, E), jnp.int32),
        ],
        scratch_shapes=[pltpu.VMEM((1, E), jnp.float32)],
    )(x2, Wg)


def _slot_body(idx_ref, rank_ref, cnt_ref, slot_ref, be_ref, val_ref):
    cnt = cnt_ref[...].astype(jnp.float32)
    pc = jnp.floor((cnt + (BLK - 1)) * (1.0 / BLK)) * BLK
    r_i = lax.broadcasted_iota(jnp.int32, (E, E), 0)
    c_i = lax.broadcasted_iota(jnp.int32, (E, E), 1)
    upper = (r_i < c_i).astype(jnp.float32)
    pstart = lax.dot_general(pc, upper, (((1,), (0,)), ((), ())),
                             preferred_element_type=jnp.float32)
    cols = lax.broadcasted_iota(jnp.int32, (NTOK, E), 1)
    oh0 = (cols == idx_ref[:, 0:1]).astype(jnp.float32)
    oh1 = (cols == idx_ref[:, 1:2]).astype(jnp.float32)
    s0 = lax.dot_general(oh0, pstart, (((1,), (1,)), ((), ())),
                         preferred_element_type=jnp.float32)
    s1 = lax.dot_general(oh1, pstart, (((1,), (1,)), ((), ())),
                         preferred_element_type=jnp.float32)
    base = jnp.concatenate([s0, s1], axis=1)
    slot_ref[...] = base.astype(jnp.int32) + rank_ref[...]

    eye = (r_i == c_i).astype(jnp.float32)
    strict = (r_i > c_i).astype(jnp.float32)
    incl = (r_i >= c_i).astype(jnp.float32)
    pad_cum_t = lax.dot_general(incl, pc, (((1,), (1,)), ((), ())),
                                preferred_element_type=jnp.float32)
    pstart_t = lax.dot_general(strict, pc, (((1,), (1,)), ((), ())),
                               preferred_element_type=jnp.float32)
    cnt_t = lax.dot_general(eye, cnt, (((1,), (1,)), ((), ())),
                            preferred_element_type=jnp.float32)
    realend_t = pstart_t + cnt_t
    b_off = (lax.broadcasted_iota(jnp.int32, (E, NB), 1)
             * BLK).astype(jnp.float32)
    eid_f = jnp.sum((pad_cum_t <= b_off).astype(jnp.float32),
                    axis=0, keepdims=True)
    val_f = jnp.sum(((pstart_t <= b_off) & (b_off < realend_t))
                    .astype(jnp.float32), axis=0, keepdims=True)
    be_ref[...] = jnp.minimum(eid_f, float(E - 1)).astype(jnp.int32)
    val_ref[...] = val_f.astype(jnp.int32)


def _slots(idx, rank, cnt):
    return pl.pallas_call(
        _slot_body,
        grid=(1,),
        in_specs=[
            pl.BlockSpec((NTOK, TOP_K), lambda i: (0, 0)),
            pl.BlockSpec((NTOK, TOP_K), lambda i: (0, 0)),
            pl.BlockSpec((1, E), lambda i: (0, 0)),
        ],
        out_specs=[
            pl.BlockSpec((NTOK, TOP_K), lambda i: (0, 0)),
            pl.BlockSpec((1, NB), lambda i: (0, 0)),
            pl.BlockSpec((1, NB), lambda i: (0, 0)),
        ],
        out_shape=[
            jax.ShapeDtypeStruct((NTOK, TOP_K), jnp.int32),
            jax.ShapeDtypeStruct((1, NB), jnp.int32),
            jax.ShapeDtypeStruct((1, NB), jnp.int32),
        ],
    )(idx, rank, cnt)


_SC_CH = 64


def _iota16(base):
    return lax.iota(jnp.int32, 16) + base


def _sc_dispatch(x2, slot):
    na = NTOK * TOP_K
    info = plsc.get_sparse_core_info()
    nw = info.num_cores * info.num_subcores
    per_w = na // nw
    steps = per_w // _SC_CH
    mesh = plsc.VectorSubcoreMesh(core_axis_name="c", subcore_axis_name="s")

    @functools.partial(
        pl.kernel,
        mesh=mesh,
        out_type=jax.ShapeDtypeStruct((LPAD, DIM), jnp.float32),
        scratch_types=[
            pltpu.VMEM((_SC_CH,), jnp.int32),
            pltpu.VMEM((_SC_CH,), jnp.int32),
            pltpu.VMEM((_SC_CH, DIM), jnp.float32),
            pltpu.SemaphoreType.DMA,
            pltpu.SemaphoreType.DMA,
        ],
    )
    def dispatch_k(x_hbm, slot_hbm, xs_hbm, sv, tok_v, rows_v, s0, s1):
        wid = lax.axis_index("s") * info.num_cores + lax.axis_index("c")
        base = wid * per_w

        def body(i, carry):
            aoff = base + i * _SC_CH
            pltpu.sync_copy(slot_hbm.at[pl.ds(aoff, _SC_CH)], sv)
            for b in range(_SC_CH // 16):
                j = _iota16(aoff + 16 * b)
                tok_v[pl.ds(16 * b, 16)] = lax.shift_right_logical(j, 1)
            pltpu.async_copy(x_hbm.at[tok_v], rows_v, s0).wait()
            pltpu.async_copy(rows_v, xs_hbm.at[sv], s1).wait()
            return carry

        lax.fori_loop(0, steps, body, 0)

    return dispatch_k(x2, slot)


def _sc_combine(ys, slot):
    na = NTOK * TOP_K
    info = plsc.get_sparse_core_info()
    nw = info.num_cores * info.num_subcores
    per_w = na // nw
    steps = per_w // _SC_CH
    mesh = plsc.VectorSubcoreMesh(core_axis_name="c", subcore_axis_name="s")

    @functools.partial(
        pl.kernel,
        mesh=mesh,
        out_type=jax.ShapeDtypeStruct((TOP_K * NTOK, DIM), jnp.float32),
        scratch_types=[
            pltpu.VMEM((_SC_CH,), jnp.int32),
            pltpu.VMEM((_SC_CH,), jnp.int32),
            pltpu.VMEM((_SC_CH, DIM), jnp.float32),
            pltpu.SemaphoreType.DMA,
            pltpu.SemaphoreType.DMA,
        ],
    )
    def combine_k(ys_hbm, slot_hbm, g_hbm, sv, dst_v, rows_v, s0, s1):
        wid = lax.axis_index("s") * info.num_cores + lax.axis_index("c")
        base = wid * per_w

        def body(i, carry):
            aoff = base + i * _SC_CH
            pltpu.sync_copy(slot_hbm.at[pl.ds(aoff, _SC_CH)], sv)
            for b in range(_SC_CH // 16):
                j = _iota16(aoff + 16 * b)
                dst_v[pl.ds(16 * b, 16)] = (
                    lax.shift_right_logical(j, 1)
                    + (j & 1) * NTOK)
            pltpu.async_copy(ys_hbm.at[sv], rows_v, s0).wait()
            pltpu.async_copy(rows_v, g_hbm.at[dst_v], s1).wait()
            return carry

        lax.fori_loop(0, steps, body, 0)

    return combine_k(ys, slot)


def _group_body(eid_ref, val_ref, xs_ref, w1_ref, w3_ref, w2_ref, ys_ref):
    @pl.when(val_ref[pl.program_id(0)] == 1)
    def _():
        x = xs_ref[...].astype(jnp.bfloat16)
        w1b = w1_ref[0].astype(jnp.bfloat16)
        w3b = w3_ref[0].astype(jnp.bfloat16)
        h1 = lax.dot_general(x, w1b, (((1,), (1,)), ((), ())),
                             preferred_element_type=jnp.float32)
        h3 = lax.dot_general(x, w3b, (((1,), (1,)), ((), ())),
                             preferred_element_type=jnp.float32)
        h = ((h1 * jax.nn.sigmoid(h1)) * h3).astype(jnp.bfloat16)
        w2b = w2_ref[0].astype(jnp.bfloat16)
        ys_ref[...] = lax.dot_general(h, w2b, (((1,), (1,)), ((), ())),
                                      preferred_element_type=jnp.float32)


def _grouped_mlp(xs, w1, w2, w3, blk_eid, valid):
    grid_spec = pltpu.PrefetchScalarGridSpec(
        num_scalar_prefetch=2,
        grid=(NB,),
        in_specs=[
            pl.BlockSpec((BLK, DIM), lambda i, e, v: (i, 0)),
            pl.BlockSpec((1, INTER, DIM), lambda i, e, v: (e[i], 0, 0)),
            pl.BlockSpec((1, INTER, DIM), lambda i, e, v: (e[i], 0, 0)),
            pl.BlockSpec((1, DIM, INTER), lambda i, e, v: (e[i], 0, 0)),
        ],
        out_specs=pl.BlockSpec((BLK, DIM), lambda i, e, v: (i, 0)),
    )
    return pl.pallas_call(
        _group_body,
        grid_spec=grid_spec,
        out_shape=jax.ShapeDtypeStruct((LPAD, DIM), jnp.float32),
    )(blk_eid, valid, xs, w1, w3, w2)


def _final_body(x_ref, sw1_ref, sw3_ref, sw2_ref, g_ref, wgt_ref, out_ref):
    x = x_ref[...]
    h1 = lax.dot_general(x, sw1_ref[...], (((1,), (1,)), ((), ())),
                         preferred_element_type=jnp.float32)
    h3 = lax.dot_general(x, sw3_ref[...], (((1,), (1,)), ((), ())),
                         preferred_element_type=jnp.float32)
    h = (h1 * jax.nn.sigmoid(h1)) * h3
    z = lax.dot_general(h, sw2_ref[...], (((1,), (1,)), ((), ())),
                        preferred_element_type=jnp.float32)
    w0 = wgt_ref[:, 0:1]
    w1c = wgt_ref[:, 1:2]
    out_ref[...] = z + w0 * g_ref[0] + w1c * g_ref[1]


def _final(x2, sw1, sw2, sw3, garr, wgt):
    n = x2.shape[0]
    return pl.pallas_call(
        _final_body,
        grid=(n // GATE_BLK,),
        in_specs=[
            pl.BlockSpec((GATE_BLK, DIM), lambda i: (i, 0)),
            pl.BlockSpec((INTER, DIM), lambda i: (0, 0)),
            pl.BlockSpec((INTER, DIM), lambda i: (0, 0)),
            pl.BlockSpec((DIM, INTER), lambda i: (0, 0)),
            pl.BlockSpec((TOP_K, GATE_BLK, DIM), lambda i: (0, i, 0)),
            pl.BlockSpec((GATE_BLK, TOP_K), lambda i: (i, 0)),
        ],
        out_specs=pl.BlockSpec((GATE_BLK, DIM), lambda i: (i, 0)),
        out_shape=jax.ShapeDtypeStruct((n, DIM), jnp.float32),
    )(x2, sw1, sw3, sw2, garr, wgt)


def kernel(x, Wg, w1, w2, w3, sw1, sw2, sw3):
    shape = x.shape
    x2 = x.reshape(-1, DIM)

    idx, wgt, rank, cnt = _gate(x2, Wg)
    slot, be2, val2 = _slots(idx, rank, cnt)
    blk_eid = be2.reshape(NB)
    valid = val2.reshape(NB)

    slot_flat = slot.reshape(-1)
    xs = _sc_dispatch(x2, slot_flat)
    ys = _grouped_mlp(xs, w1, w2, w3, blk_eid, valid)
    garr = _sc_combine(ys, slot_flat).reshape(TOP_K, NTOK, DIM)
    out = _final(x2, sw1, sw2, sw3, garr, wgt)
    return out.reshape(shape)
